# baseline (speedup 1.0000x reference)
import functools

import jax
import jax.numpy as jnp
from jax import lax
from jax.experimental import pallas as pl
from jax.experimental.pallas import tpu as pltpu
from jax.experimental.pallas import tpu_sc as plsc

NC, NS = 2, 16
NW = NC * NS
CR = 64
FEATS = 26
CI = CR * 32
NB = 2


def _sc_reorder(obs2, n_rows):
    rows_per_w = n_rows // NW
    steps = rows_per_w // CR
    mesh = plsc.VectorSubcoreMesh(core_axis_name="c", subcore_axis_name="s")

    @functools.partial(
        pl.kernel,
        out_type=jax.ShapeDtypeStruct((n_rows * 32,), jnp.int32),
        mesh=mesh,
        scratch_types=[
            pltpu.VMEM((CR, FEATS), jnp.int32),
            pltpu.VMEM((CI,), jnp.int32),
        ],
        compiler_params=pltpu.CompilerParams(
            use_tc_tiling_on_sc=True, needs_layout_passes=False
        ),
    )
    def reorder_kernel(obs_hbm, out_hbm, tbuf, ic_v):
        wid = lax.axis_index("s") * NC + lax.axis_index("c")
        c0 = wid * steps

        lane = lax.iota(jnp.int32, 16)

        def body(t, carry):
            c = c0 + t
            pltpu.sync_copy(obs_hbm.at[pl.ds(c * CR, CR)], tbuf)
            for jj in range(32):
                fl = lane + 16 * jj
                ri = fl // 8
                s = fl - 8 * ri
                for g in range(4):
                    ci = jnp.minimum(s + 8 * g, FEATS - 1)
                    ic_v[pl.ds(512 * g + 16 * jj, 16)] = plsc.load_gather(
                        tbuf, [ri, ci]
                    )
            pltpu.sync_copy(ic_v, out_hbm.at[pl.ds(c * CI, CI)])
            return carry

        lax.fori_loop(0, steps, body, 0)

    return reorder_kernel(obs2)


def _sc_gather(table, idxr, n_rows, d):
    rows_per_w = n_rows // NW
    steps = rows_per_w // CR
    mesh = plsc.VectorSubcoreMesh(core_axis_name="c", subcore_axis_name="s")

    @functools.partial(
        pl.kernel,
        out_type=jax.ShapeDtypeStruct((4, n_rows * 8, d), jnp.float32),
        mesh=mesh,
        scratch_types=[
            pltpu.VMEM((CI,), jnp.int32),
            pltpu.VMEM((CI,), jnp.int32),
            pltpu.VMEM((CI, 16), jnp.float32),
            pltpu.VMEM((CI, 16), jnp.float32),
            pltpu.SemaphoreType.DMA,
            pltpu.SemaphoreType.DMA,
            pltpu.SemaphoreType.DMA,
        ],
        compiler_params=pltpu.CompilerParams(
            use_tc_tiling_on_sc=False, needs_layout_passes=False
        ),
    )
    def gather_kernel(table_hbm, idx_hbm, out_hbm,
                      ic_a, ic_b, buf_a, buf_b,
                      sem_g, sem_wa, sem_wb):
        wid = lax.axis_index("s") * NC + lax.axis_index("c")
        c0 = wid * steps

        bufs = ((ic_a, buf_a, sem_wa), (ic_b, buf_b, sem_wb))

        def body(t2, carry):
            for p, (ic_v, buf_v, sem_w) in enumerate(bufs):
                c = c0 + 2 * t2 + p
                pltpu.sync_copy(idx_hbm.at[pl.ds(c * CI, CI)], ic_v)

                @pl.when(t2 > 0)
                def _():
                    for g in range(4):
                        pltpu.make_async_copy(
                            buf_v.at[pl.ds(512 * g, 512)],
                            out_hbm.at[g, pl.ds(c * 512, 512)],
                            sem_w,
                        ).wait()

                copies = [
                    pltpu.async_copy(
                        table_hbm.at[ic_v.at[pl.ds(128 * s, 128)]],
                        buf_v.at[pl.ds(128 * s, 128)],
                        sem_g,
                    )
                    for s in range(CI // 128)
                ]
                for cp in copies:
                    cp.wait()
                for g in range(4):
                    pltpu.async_copy(
                        buf_v.at[pl.ds(512 * g, 512)],
                        out_hbm.at[g, pl.ds(c * 512, 512)],
                        sem_w,
                    )
            return carry

        lax.fori_loop(0, steps // 2, body, 0)
        for ic_v, buf_v, sem_w in bufs:
            for g in range(4):
                pltpu.make_async_copy(
                    buf_v.at[pl.ds(512 * g, 512)],
                    out_hbm.at[g, pl.ds(c0 * 512, 512)],
                    sem_w,
                ).wait()

    return gather_kernel(table, idxr)


def _tc_matmul(x3, w4, b, n_total, grid_off, alias_out=None):
    half = x3.shape[1]
    out_dim = w4.shape[2]
    bm = 1024

    def mm_kernel(x_ref, w_ref, b_ref, *rest):
        o_ref = rest[-1]
        acc = lax.dot_general(
            x_ref[0], w_ref[0], (((1,), (0,)), ((), ())),
            preferred_element_type=jnp.float32,
        )
        for g in range(1, 4):
            acc += lax.dot_general(
                x_ref[g], w_ref[g], (((1,), (0,)), ((), ())),
                preferred_element_type=jnp.float32,
            )
        o_ref[...] = acc + b_ref[...]

    in_specs = [
        pl.BlockSpec((4, bm, 128), lambda i: (0, i, 0)),
        pl.BlockSpec((4, 128, out_dim), lambda i: (0, 0, 0)),
        pl.BlockSpec((1, out_dim), lambda i: (0, 0)),
    ]
    args = [x3, w4, b.reshape(1, out_dim)]
    kwargs = {}
    if alias_out is not None:
        in_specs.append(pl.BlockSpec(memory_space=pl.ANY))
        args.append(alias_out)
        kwargs["input_output_aliases"] = {3: 0}

    return pl.pallas_call(
        mm_kernel,
        grid=(half // bm,),
        in_specs=in_specs,
        out_specs=pl.BlockSpec((bm, out_dim), lambda i, o=grid_off: (i + o, 0)),
        out_shape=jax.ShapeDtypeStruct((n_total, out_dim), jnp.float32),
        **kwargs,
    )(*args)


def kernel(obs, table, W, b):
    batch, context_len, n_agents, features = obs.shape
    n = batch * context_len * n_agents
    d = table.shape[1]
    out_dim = W.shape[0]
    half = n // 2

    obs2 = obs.reshape(n, features)
    idxr = _sc_reorder(obs2, n)

    wp = jnp.pad(W, ((0, 0), (0, 32 * d - features * d)))
    w4 = wp.reshape(out_dim, 4, 8 * d).transpose(1, 2, 0)

    gx1 = _sc_gather(table, idxr[: half * 32], half, d)
    gx2 = _sc_gather(table, idxr[half * 32:], half, d)
    x3a = gx1.reshape(4, half, 8 * d)
    x3b = gx2.reshape(4, half, 8 * d)

    o1 = _tc_matmul(x3a, w4, b, n, 0)
    out = _tc_matmul(x3b, w4, b, n, half // 1024, alias_out=o1)
    return out.reshape(batch, context_len, n_agents, out_dim)


# speedup vs baseline: 1.0323x; 1.0323x over previous
import functools

import jax
import jax.numpy as jnp
from jax import lax
from jax.experimental import pallas as pl
from jax.experimental.pallas import tpu as pltpu
from jax.experimental.pallas import tpu_sc as plsc

NC, NS = 2, 16
NW = NC * NS
CR = 64
FEATS = 26
CI = CR * 32
NB = 2


def _sc_reorder(obs, n_rows):
    batch = obs.shape[0]
    rows_per_b = obs.shape[1] * obs.shape[2]
    b_per_w = batch // NW
    steps = b_per_w // NB
    gpc = NB * rows_per_b // CR
    cic = gpc * CI
    mesh = plsc.VectorSubcoreMesh(core_axis_name="c", subcore_axis_name="s")

    @functools.partial(
        pl.kernel,
        out_type=jax.ShapeDtypeStruct((n_rows * 32,), jnp.int32),
        mesh=mesh,
        scratch_types=[
            pltpu.VMEM((NB,) + obs.shape[1:], jnp.int32),
            pltpu.VMEM((cic,), jnp.int32),
        ],
        compiler_params=pltpu.CompilerParams(
            use_tc_tiling_on_sc=True, needs_layout_passes=False
        ),
    )
    def reorder_kernel(obs_hbm, out_hbm, tbuf, ic_v):
        wid = lax.axis_index("s") * NC + lax.axis_index("c")
        b0 = wid * b_per_w
        g0 = wid * (b_per_w * rows_per_b // CR)

        lane = lax.iota(jnp.int32, 16)

        def body(t, carry):
            pltpu.sync_copy(obs_hbm.at[pl.ds(b0 + NB * t, NB)], tbuf)
            for k in range(gpc):
                for jj in range(32):
                    fl = lane + 16 * jj
                    rl = fl // 8 + CR * k
                    bi = rl // rows_per_b
                    rem = rl - rows_per_b * bi
                    li = rem // 8
                    ai = rem - 8 * li
                    s = fl - 8 * (fl // 8)
                    for g in range(4):
                        ci = jnp.minimum(s + 8 * g, FEATS - 1)
                        ic_v[pl.ds(CI * k + 512 * g + 16 * jj, 16)] = (
                            plsc.load_gather(tbuf, [bi, li, ai, ci])
                        )
            pltpu.sync_copy(
                ic_v, out_hbm.at[pl.ds(CI * (g0 + gpc * t), cic)]
            )
            return carry

        lax.fori_loop(0, steps, body, 0)

    return reorder_kernel(obs)


def _sc_gather(table, idxr, n_rows, d):
    rows_per_w = n_rows // NW
    steps = rows_per_w // CR
    mesh = plsc.VectorSubcoreMesh(core_axis_name="c", subcore_axis_name="s")

    @functools.partial(
        pl.kernel,
        out_type=jax.ShapeDtypeStruct((4, n_rows * 8, d), jnp.float32),
        mesh=mesh,
        scratch_types=[
            pltpu.VMEM((CI,), jnp.int32),
            pltpu.VMEM((CI,), jnp.int32),
            pltpu.VMEM((CI, 16), jnp.float32),
            pltpu.VMEM((CI, 16), jnp.float32),
            pltpu.SemaphoreType.DMA,
            pltpu.SemaphoreType.DMA,
            pltpu.SemaphoreType.DMA,
        ],
        compiler_params=pltpu.CompilerParams(
            use_tc_tiling_on_sc=False, needs_layout_passes=False
        ),
    )
    def gather_kernel(table_hbm, idx_hbm, out_hbm,
                      ic_a, ic_b, buf_a, buf_b,
                      sem_g, sem_wa, sem_wb):
        wid = lax.axis_index("s") * NC + lax.axis_index("c")
        c0 = wid * steps

        bufs = ((ic_a, buf_a, sem_wa), (ic_b, buf_b, sem_wb))

        def body(t2, carry):
            for p, (ic_v, buf_v, sem_w) in enumerate(bufs):
                c = c0 + 2 * t2 + p
                pltpu.sync_copy(idx_hbm.at[pl.ds(c * CI, CI)], ic_v)

                @pl.when(t2 > 0)
                def _():
                    for g in range(4):
                        pltpu.make_async_copy(
                            buf_v.at[pl.ds(512 * g, 512)],
                            out_hbm.at[g, pl.ds(c * 512, 512)],
                            sem_w,
                        ).wait()

                copies = [
                    pltpu.async_copy(
                        table_hbm.at[ic_v.at[pl.ds(128 * s, 128)]],
                        buf_v.at[pl.ds(128 * s, 128)],
                        sem_g,
                    )
                    for s in range(CI // 128)
                ]
                for cp in copies:
                    cp.wait()
                for g in range(4):
                    pltpu.async_copy(
                        buf_v.at[pl.ds(512 * g, 512)],
                        out_hbm.at[g, pl.ds(c * 512, 512)],
                        sem_w,
                    )
            return carry

        lax.fori_loop(0, steps // 2, body, 0)
        for ic_v, buf_v, sem_w in bufs:
            for g in range(4):
                pltpu.make_async_copy(
                    buf_v.at[pl.ds(512 * g, 512)],
                    out_hbm.at[g, pl.ds(c0 * 512, 512)],
                    sem_w,
                ).wait()

    return gather_kernel(table, idxr)


def _tc_matmul(x3, w4, b, n_total, grid_off, alias_out=None):
    half = x3.shape[1]
    out_dim = w4.shape[2]
    bm = 2048

    def mm_kernel(x_ref, w_ref, b_ref, *rest):
        o_ref = rest[-1]
        acc = lax.dot_general(
            x_ref[0], w_ref[0], (((1,), (0,)), ((), ())),
            preferred_element_type=jnp.float32,
        )
        for g in range(1, 4):
            acc += lax.dot_general(
                x_ref[g], w_ref[g], (((1,), (0,)), ((), ())),
                preferred_element_type=jnp.float32,
            )
        o_ref[...] = acc + b_ref[...]

    in_specs = [
        pl.BlockSpec((4, bm, 128), lambda i: (0, i, 0)),
        pl.BlockSpec((4, 128, out_dim), lambda i: (0, 0, 0)),
        pl.BlockSpec((1, out_dim), lambda i: (0, 0)),
    ]
    args = [x3, w4, b.reshape(1, out_dim)]
    kwargs = {}
    if alias_out is not None:
        in_specs.append(pl.BlockSpec(memory_space=pl.ANY))
        args.append(alias_out)
        kwargs["input_output_aliases"] = {3: 0}

    return pl.pallas_call(
        mm_kernel,
        grid=(half // bm,),
        in_specs=in_specs,
        out_specs=pl.BlockSpec((bm, out_dim), lambda i, o=grid_off: (i + o, 0)),
        out_shape=jax.ShapeDtypeStruct((n_total, out_dim), jnp.float32),
        **kwargs,
    )(*args)


def kernel(obs, table, W, b):
    batch, context_len, n_agents, features = obs.shape
    n = batch * context_len * n_agents
    d = table.shape[1]
    out_dim = W.shape[0]
    half = n // 2

    idxr = _sc_reorder(obs, n)

    wp = jnp.pad(W, ((0, 0), (0, 32 * d - features * d)))
    w4 = wp.reshape(out_dim, 4, 8 * d).transpose(1, 2, 0)

    gx1 = _sc_gather(table, idxr[: half * 32], half, d)
    gx2 = _sc_gather(table, idxr[half * 32:], half, d)
    x3a = gx1.reshape(4, half, 8 * d)
    x3b = gx2.reshape(4, half, 8 * d)

    o1 = _tc_matmul(x3a, w4, b, n, 0)
    out = _tc_matmul(x3b, w4, b, n, half // 2048, alias_out=o1)
    return out.reshape(batch, context_len, n_agents, out_dim)


# speedup vs baseline: 1.0408x; 1.0083x over previous
import functools

import jax
import jax.numpy as jnp
from jax import lax
from jax.experimental import pallas as pl
from jax.experimental.pallas import tpu as pltpu
from jax.experimental.pallas import tpu_sc as plsc

NC, NS = 2, 16
NW = NC * NS
CR = 64
FEATS = 26
CI = CR * 32
NB = 2


def _sc_reorder(obs, n_rows):
    batch = obs.shape[0]
    rows_per_b = obs.shape[1] * obs.shape[2]
    b_per_w = batch // NW
    steps = b_per_w // NB
    gpc = NB * rows_per_b // CR
    cic = gpc * CI
    mesh = plsc.VectorSubcoreMesh(core_axis_name="c", subcore_axis_name="s")

    @functools.partial(
        pl.kernel,
        out_type=jax.ShapeDtypeStruct((n_rows * 32,), jnp.int32),
        mesh=mesh,
        scratch_types=[
            pltpu.VMEM((NB,) + obs.shape[1:], jnp.int32),
            pltpu.VMEM((cic,), jnp.int32),
        ],
        compiler_params=pltpu.CompilerParams(
            use_tc_tiling_on_sc=True, needs_layout_passes=False
        ),
    )
    def reorder_kernel(obs_hbm, out_hbm, tbuf, ic_v):
        wid = lax.axis_index("s") * NC + lax.axis_index("c")
        b0 = wid * b_per_w
        g0 = wid * (b_per_w * rows_per_b // CR)

        lane = lax.iota(jnp.int32, 16)

        def body(t, carry):
            pltpu.sync_copy(obs_hbm.at[pl.ds(b0 + NB * t, NB)], tbuf)
            for k in range(gpc):
                for jj in range(32):
                    fl = lane + 16 * jj
                    rl = fl // 8 + CR * k
                    bi = rl // rows_per_b
                    rem = rl - rows_per_b * bi
                    li = rem // 8
                    ai = rem - 8 * li
                    s = fl - 8 * (fl // 8)
                    for g in range(4):
                        ci = jnp.minimum(s + 8 * g, FEATS - 1)
                        ic_v[pl.ds(CI * k + 512 * g + 16 * jj, 16)] = (
                            plsc.load_gather(tbuf, [bi, li, ai, ci])
                        )
            pltpu.sync_copy(
                ic_v, out_hbm.at[pl.ds(CI * (g0 + gpc * t), cic)]
            )
            return carry

        lax.fori_loop(0, steps, body, 0)

    return reorder_kernel(obs)


def _sc_gather(table, idxr, n_rows, d):
    rows_per_w = n_rows // NW
    steps = rows_per_w // CR
    mesh = plsc.VectorSubcoreMesh(core_axis_name="c", subcore_axis_name="s")

    @functools.partial(
        pl.kernel,
        out_type=jax.ShapeDtypeStruct((4, n_rows * 8, d), jnp.float32),
        mesh=mesh,
        scratch_types=[
            pltpu.VMEM((CI,), jnp.int32),
            pltpu.VMEM((CI,), jnp.int32),
            pltpu.VMEM((CI, 16), jnp.float32),
            pltpu.VMEM((CI, 16), jnp.float32),
            pltpu.SemaphoreType.DMA,
            pltpu.SemaphoreType.DMA,
            pltpu.SemaphoreType.DMA,
        ],
        compiler_params=pltpu.CompilerParams(
            use_tc_tiling_on_sc=False, needs_layout_passes=False
        ),
    )
    def gather_kernel(table_hbm, idx_hbm, out_hbm,
                      ic_a, ic_b, buf_a, buf_b,
                      sem_g, sem_wa, sem_wb):
        wid = lax.axis_index("s") * NC + lax.axis_index("c")
        c0 = wid * steps

        bufs = ((ic_a, buf_a, sem_wa), (ic_b, buf_b, sem_wb))

        def body(t2, carry):
            for p, (ic_v, buf_v, sem_w) in enumerate(bufs):
                c = c0 + 2 * t2 + p
                pltpu.sync_copy(idx_hbm.at[pl.ds(c * CI, CI)], ic_v)

                @pl.when(t2 > 0)
                def _():
                    for g in range(4):
                        pltpu.make_async_copy(
                            buf_v.at[pl.ds(512 * g, 512)],
                            out_hbm.at[g, pl.ds(c * 512, 512)],
                            sem_w,
                        ).wait()

                copies = [
                    pltpu.async_copy(
                        table_hbm.at[ic_v.at[pl.ds(128 * s, 128)]],
                        buf_v.at[pl.ds(128 * s, 128)],
                        sem_g,
                    )
                    for s in range(CI // 128)
                ]
                for cp in copies:
                    cp.wait()
                for g in range(4):
                    pltpu.async_copy(
                        buf_v.at[pl.ds(512 * g, 512)],
                        out_hbm.at[g, pl.ds(c * 512, 512)],
                        sem_w,
                    )
            return carry

        lax.fori_loop(0, steps // 2, body, 0)
        for ic_v, buf_v, sem_w in bufs:
            for g in range(4):
                pltpu.make_async_copy(
                    buf_v.at[pl.ds(512 * g, 512)],
                    out_hbm.at[g, pl.ds(c0 * 512, 512)],
                    sem_w,
                ).wait()

    return gather_kernel(table, idxr)


def _tc_matmul(x3, w4, b, n_total, grid_off, alias_out=None):
    half = x3.shape[1]
    out_dim = w4.shape[2]
    bm = 4096

    def mm_kernel(x_ref, w_ref, b_ref, *rest):
        o_ref = rest[-1]
        acc = lax.dot_general(
            x_ref[0], w_ref[0], (((1,), (0,)), ((), ())),
            preferred_element_type=jnp.float32,
        )
        for g in range(1, 4):
            acc += lax.dot_general(
                x_ref[g], w_ref[g], (((1,), (0,)), ((), ())),
                preferred_element_type=jnp.float32,
            )
        o_ref[...] = acc + b_ref[...]

    in_specs = [
        pl.BlockSpec((4, bm, 128), lambda i: (0, i, 0)),
        pl.BlockSpec((4, 128, out_dim), lambda i: (0, 0, 0)),
        pl.BlockSpec((1, out_dim), lambda i: (0, 0)),
    ]
    args = [x3, w4, b.reshape(1, out_dim)]
    kwargs = {}
    if alias_out is not None:
        in_specs.append(pl.BlockSpec(memory_space=pl.ANY))
        args.append(alias_out)
        kwargs["input_output_aliases"] = {3: 0}

    return pl.pallas_call(
        mm_kernel,
        grid=(half // bm,),
        in_specs=in_specs,
        out_specs=pl.BlockSpec((bm, out_dim), lambda i, o=grid_off: (i + o, 0)),
        out_shape=jax.ShapeDtypeStruct((n_total, out_dim), jnp.float32),
        compiler_params=pltpu.CompilerParams(vmem_limit_bytes=100 * 1024 * 1024),
        **kwargs,
    )(*args)


def kernel(obs, table, W, b):
    batch, context_len, n_agents, features = obs.shape
    n = batch * context_len * n_agents
    d = table.shape[1]
    out_dim = W.shape[0]
    half = n // 2

    idxr = _sc_reorder(obs, n)

    wp = jnp.pad(W, ((0, 0), (0, 32 * d - features * d)))
    w4 = wp.reshape(out_dim, 4, 8 * d).transpose(1, 2, 0)

    gx1 = _sc_gather(table, idxr[: half * 32], half, d)
    gx2 = _sc_gather(table, idxr[half * 32:], half, d)
    x3a = gx1.reshape(4, half, 8 * d)
    x3b = gx2.reshape(4, half, 8 * d)

    o1 = _tc_matmul(x3a, w4, b, n, 0)
    out = _tc_matmul(x3b, w4, b, n, half // 4096, alias_out=o1)
    return out.reshape(batch, context_len, n_agents, out_dim)


# speedup vs baseline: 1.0590x; 1.0175x over previous
import functools

import jax
import jax.numpy as jnp
from jax import lax
from jax.experimental import pallas as pl
from jax.experimental.pallas import tpu as pltpu
from jax.experimental.pallas import tpu_sc as plsc

NC, NS = 2, 16
NW = NC * NS
CR = 80
PG = CR * 8
FEATS = 26
CI = CR * 32
NB = 2


def _sc_reorder(obs, n_rows):
    batch = obs.shape[0]
    rows_per_b = obs.shape[1] * obs.shape[2]
    b_per_w = batch // NW
    steps = b_per_w // NB
    gpc = NB * rows_per_b // CR
    cic = gpc * CI
    mesh = plsc.VectorSubcoreMesh(core_axis_name="c", subcore_axis_name="s")

    @functools.partial(
        pl.kernel,
        out_type=jax.ShapeDtypeStruct((n_rows * 32,), jnp.int32),
        mesh=mesh,
        scratch_types=[
            pltpu.VMEM((NB,) + obs.shape[1:], jnp.int32),
            pltpu.VMEM((cic,), jnp.int32),
        ],
        compiler_params=pltpu.CompilerParams(
            use_tc_tiling_on_sc=True, needs_layout_passes=False
        ),
    )
    def reorder_kernel(obs_hbm, out_hbm, tbuf, ic_v):
        wid = lax.axis_index("s") * NC + lax.axis_index("c")
        b0 = wid * b_per_w
        g0 = wid * (b_per_w * rows_per_b // CR)

        lane = lax.iota(jnp.int32, 16)

        def body(t, carry):
            pltpu.sync_copy(obs_hbm.at[pl.ds(b0 + NB * t, NB)], tbuf)
            for k in range(gpc):
                for jj in range(PG // 16):
                    fl = lane + 16 * jj
                    rl = fl // 8 + CR * k
                    bi = rl // rows_per_b
                    rem = rl - rows_per_b * bi
                    li = rem // 8
                    ai = rem - 8 * li
                    s = fl - 8 * (fl // 8)
                    for g in range(4):
                        ci = jnp.minimum(s + 8 * g, FEATS - 1)
                        ic_v[pl.ds(CI * k + PG * g + 16 * jj, 16)] = (
                            plsc.load_gather(tbuf, [bi, li, ai, ci])
                        )
            pltpu.sync_copy(
                ic_v, out_hbm.at[pl.ds(CI * (g0 + gpc * t), cic)]
            )
            return carry

        lax.fori_loop(0, steps, body, 0)

    return reorder_kernel(obs)


def _sc_gather(table, idxr, n_rows, d):
    rows_per_w = n_rows // NW
    steps = rows_per_w // CR
    mesh = plsc.VectorSubcoreMesh(core_axis_name="c", subcore_axis_name="s")

    @functools.partial(
        pl.kernel,
        out_type=jax.ShapeDtypeStruct((4, n_rows * 8, d), jnp.float32),
        mesh=mesh,
        scratch_types=[
            pltpu.VMEM((CI,), jnp.int32),
            pltpu.VMEM((CI,), jnp.int32),
            pltpu.VMEM((CI, 16), jnp.float32),
            pltpu.VMEM((CI, 16), jnp.float32),
            pltpu.SemaphoreType.DMA,
            pltpu.SemaphoreType.DMA,
            pltpu.SemaphoreType.DMA,
        ],
        compiler_params=pltpu.CompilerParams(
            use_tc_tiling_on_sc=False, needs_layout_passes=False
        ),
    )
    def gather_kernel(table_hbm, idx_hbm, out_hbm,
                      ic_a, ic_b, buf_a, buf_b,
                      sem_g, sem_wa, sem_wb):
        wid = lax.axis_index("s") * NC + lax.axis_index("c")
        c0 = wid * steps

        bufs = ((ic_a, buf_a, sem_wa), (ic_b, buf_b, sem_wb))

        def body(t2, carry):
            for p, (ic_v, buf_v, sem_w) in enumerate(bufs):
                c = c0 + 2 * t2 + p
                pltpu.sync_copy(idx_hbm.at[pl.ds(c * CI, CI)], ic_v)

                @pl.when(t2 > 0)
                def _():
                    for g in range(4):
                        pltpu.make_async_copy(
                            buf_v.at[pl.ds(PG * g, PG)],
                            out_hbm.at[g, pl.ds(c * PG, PG)],
                            sem_w,
                        ).wait()

                copies = [
                    pltpu.async_copy(
                        table_hbm.at[ic_v.at[pl.ds(128 * s, 128)]],
                        buf_v.at[pl.ds(128 * s, 128)],
                        sem_g,
                    )
                    for s in range(CI // 128)
                ]
                for cp in copies:
                    cp.wait()
                for g in range(4):
                    pltpu.async_copy(
                        buf_v.at[pl.ds(PG * g, PG)],
                        out_hbm.at[g, pl.ds(c * PG, PG)],
                        sem_w,
                    )
            return carry

        lax.fori_loop(0, steps // 2, body, 0)
        for ic_v, buf_v, sem_w in bufs:
            for g in range(4):
                pltpu.make_async_copy(
                    buf_v.at[pl.ds(PG * g, PG)],
                    out_hbm.at[g, pl.ds(c0 * PG, PG)],
                    sem_w,
                ).wait()

    return gather_kernel(table, idxr)


def _tc_matmul(x3, w4, b, n_total, grid_off, alias_out=None):
    half = x3.shape[1]
    out_dim = w4.shape[2]
    bm = 4096

    def mm_kernel(x_ref, w_ref, b_ref, *rest):
        o_ref = rest[-1]
        acc = lax.dot_general(
            x_ref[0], w_ref[0], (((1,), (0,)), ((), ())),
            preferred_element_type=jnp.float32,
        )
        for g in range(1, 4):
            acc += lax.dot_general(
                x_ref[g], w_ref[g], (((1,), (0,)), ((), ())),
                preferred_element_type=jnp.float32,
            )
        o_ref[...] = acc + b_ref[...]

    in_specs = [
        pl.BlockSpec((4, bm, 128), lambda i: (0, i, 0)),
        pl.BlockSpec((4, 128, out_dim), lambda i: (0, 0, 0)),
        pl.BlockSpec((1, out_dim), lambda i: (0, 0)),
    ]
    args = [x3, w4, b.reshape(1, out_dim)]
    kwargs = {}
    if alias_out is not None:
        in_specs.append(pl.BlockSpec(memory_space=pl.ANY))
        args.append(alias_out)
        kwargs["input_output_aliases"] = {3: 0}

    return pl.pallas_call(
        mm_kernel,
        grid=(half // bm,),
        in_specs=in_specs,
        out_specs=pl.BlockSpec((bm, out_dim), lambda i, o=grid_off: (i + o, 0)),
        out_shape=jax.ShapeDtypeStruct((n_total, out_dim), jnp.float32),
        compiler_params=pltpu.CompilerParams(vmem_limit_bytes=100 * 1024 * 1024),
        **kwargs,
    )(*args)


def kernel(obs, table, W, b):
    batch, context_len, n_agents, features = obs.shape
    n = batch * context_len * n_agents
    d = table.shape[1]
    out_dim = W.shape[0]
    half = n // 2

    idxr = _sc_reorder(obs, n)

    wp = jnp.pad(W, ((0, 0), (0, 32 * d - features * d)))
    w4 = wp.reshape(out_dim, 4, 8 * d).transpose(1, 2, 0)

    gx1 = _sc_gather(table, idxr[: half * 32], half, d)
    gx2 = _sc_gather(table, idxr[half * 32:], half, d)
    x3a = gx1.reshape(4, half, 8 * d)
    x3b = gx2.reshape(4, half, 8 * d)

    o1 = _tc_matmul(x3a, w4, b, n, 0)
    out = _tc_matmul(x3b, w4, b, n, half // 4096, alias_out=o1)
    return out.reshape(batch, context_len, n_agents, out_dim)
